# baseline (speedup 1.0000x reference)
import functools

import jax
import jax.numpy as jnp
from jax import lax
from jax.experimental import pallas as pl
from jax.experimental.pallas import tpu as pltpu
from jax.experimental.pallas import tpu_sc as plsc

N = 10000
E = 320000
D_IN = 8
DE_IN = 10
D = 128
H = 4
DH = 32
L = 2
SCALE = 1.0 / (32.0 ** 0.5)

NC = 2
NS = 16
NW = NC * NS
EW = E // NW
CH = 80
NCHUNK = EW // CH
DENW = 8
NACC = 10112
TROWS = NACC // NS


def _edge_attn_body(q_hbm, k_hbm, v_hbm, ev_hbm, src_hbm, dst_hbm,
                    omsg_hbm, oden_hbm,
                    src_v, dst_v, qrows, krows, evrows, vrows, denb,
                    accm, accd, sem1, sem2, sem3, sem4):
    c = lax.axis_index("c")
    s = lax.axis_index("s")
    wid = s * NC + c
    base = wid * EW

    zero16 = jnp.zeros((16,), jnp.float32)

    def zbody(r, carry):
        for kk in range(D // 16):
            qrows[r, pl.ds(kk * 16, 16)] = zero16
        return carry

    lax.fori_loop(0, CH, zbody, 0)
    iota16 = lax.iota(jnp.int32, 16)
    for kk in range(CH * DENW // 16):
        p = iota16 + kk * 16
        plsc.store_scatter(denb, [p // DENW, p % DENW], zero16)

    row0 = s * TROWS
    for z in range(7):
        pltpu.sync_copy(qrows, accm.at[pl.ds(row0 + z * CH, CH)])
        pltpu.sync_copy(denb, accd.at[pl.ds(row0 + z * CH, CH)])
    rem = TROWS - 7 * CH
    pltpu.sync_copy(qrows.at[pl.ds(0, rem)], accm.at[pl.ds(row0 + 7 * CH, rem)])
    pltpu.sync_copy(denb.at[pl.ds(0, rem)], accd.at[pl.ds(row0 + 7 * CH, rem)])
    plsc.subcore_barrier()

    lane15 = lax.iota(jnp.int32, 16) == 15
    full15 = jnp.full((16,), 15, jnp.int32)
    hconsts = [jnp.full((16,), h, jnp.int32) for h in range(H)]

    def chunk_body(ci, carry):
        eb = base + ci * CH
        ci1 = pltpu.async_copy(src_hbm.at[pl.ds(eb, CH)], src_v, sem3)
        ci2 = pltpu.async_copy(dst_hbm.at[pl.ds(eb, CH)], dst_v, sem4)
        ci1.wait()
        ci2.wait()
        cp1 = pltpu.async_copy(q_hbm.at[dst_v], qrows, sem1)
        cp2 = pltpu.async_copy(k_hbm.at[src_v], krows, sem2)
        cp3 = pltpu.async_copy(v_hbm.at[src_v], vrows, sem3)
        cp4 = pltpu.async_copy(ev_hbm.at[pl.ds(eb, CH)], evrows, sem4)
        cp1.wait()
        cp2.wait()
        cp3.wait()
        cp4.wait()

        def edge1(j2, carry1):
            for jj in (j2 * 2, j2 * 2 + 1):
                ph = []
                evs = []
                for cb in range(D // 16):
                    qv = qrows[jj, pl.ds(cb * 16, 16)]
                    kv = krows[jj, pl.ds(cb * 16, 16)]
                    evv = evrows[jj, pl.ds(cb * 16, 16)]
                    evs.append(evv)
                    p = qv * (kv + evv)
                    if cb % 2 == 0:
                        ph.append(p)
                    else:
                        ph[cb // 2] = ph[cb // 2] + p
                jfull = jnp.full((16,), jj, jnp.int32)
                sh = []
                for h in range(H):
                    cum = plsc.cumsum(ph[h])
                    s_h = jnp.exp(cum[full15])
                    sh.append(s_h)
                    plsc.store_scatter(denb, [jfull, hconsts[h]], s_h, mask=lane15)
                for cb in range(D // 16):
                    vv = vrows[jj, pl.ds(cb * 16, 16)]
                    qrows[jj, pl.ds(cb * 16, 16)] = sh[cb // 2] * (vv + evs[cb])
            return carry1

        lax.fori_loop(0, CH // 2, edge1, 0)

        pltpu.sync_copy(qrows, accm.at[dst_v], add=True)
        pltpu.sync_copy(denb, accd.at[dst_v], add=True)
        return carry

    lax.fori_loop(0, NCHUNK, chunk_body, 0)
    plsc.subcore_barrier()
    pltpu.sync_copy(accm.at[pl.ds(row0, TROWS)],
                    omsg_hbm.at[c, pl.ds(row0, TROWS)])
    pltpu.sync_copy(accd.at[pl.ds(row0, TROWS)],
                    oden_hbm.at[c, pl.ds(row0, TROWS)])


def _edge_attn(q, k, v, ev, src, dst):
    mesh = plsc.VectorSubcoreMesh(core_axis_name="c", subcore_axis_name="s")
    f = pl.kernel(
        _edge_attn_body,
        mesh=mesh,
        out_type=[
            jax.ShapeDtypeStruct((NC, NACC, D), jnp.float32),
            jax.ShapeDtypeStruct((NC, NACC, DENW), jnp.float32),
        ],
        compiler_params=pltpu.CompilerParams(use_tc_tiling_on_sc=False,
                                             needs_layout_passes=False),
        scratch_types=[
            pltpu.VMEM((CH,), jnp.int32),
            pltpu.VMEM((CH,), jnp.int32),
            pltpu.VMEM((CH, D), jnp.float32),
            pltpu.VMEM((CH, D), jnp.float32),
            pltpu.VMEM((CH, D), jnp.float32),
            pltpu.VMEM((CH, D), jnp.float32),
            pltpu.VMEM((CH, DENW), jnp.float32),
            pltpu.VMEM_SHARED((NACC, D), jnp.float32),
            pltpu.VMEM_SHARED((NACC, DENW), jnp.float32),
            pltpu.SemaphoreType.DMA,
            pltpu.SemaphoreType.DMA,
            pltpu.SemaphoreType.DMA,
            pltpu.SemaphoreType.DMA,
        ],
    )
    return f(q, k, v, ev, src, dst)


def _ev_body(e_ref, w_ref, b_ref, o_ref):
    o_ref[...] = (jnp.dot(e_ref[...], w_ref[0],
                          preferred_element_type=jnp.float32)
                  + b_ref[0])[None]


def _ev_proj(e_attr, w_ev, b_ev):
    BE = 2000
    return pl.pallas_call(
        _ev_body,
        grid=(L, E // BE),
        in_specs=[
            pl.BlockSpec((BE, DE_IN), lambda l, i: (i, 0)),
            pl.BlockSpec((1, DE_IN, D), lambda l, i: (l, 0, 0)),
            pl.BlockSpec((1, 1, D), lambda l, i: (l, 0, 0)),
        ],
        out_specs=pl.BlockSpec((1, BE, D), lambda l, i: (l, i, 0)),
        out_shape=jax.ShapeDtypeStruct((L, E, D), jnp.float32),
    )(e_attr, w_ev, b_ev)


def _embed_body(x_ref, w_ref, b_ref, o_ref):
    o_ref[...] = jnp.dot(x_ref[...], w_ref[...],
                         preferred_element_type=jnp.float32) + b_ref[...]


def _embed(x, w, b):
    BN = 2000
    return pl.pallas_call(
        _embed_body,
        grid=(N // BN,),
        in_specs=[
            pl.BlockSpec((BN, D_IN), lambda i: (i, 0)),
            pl.BlockSpec((D_IN, D), lambda i: (0, 0)),
            pl.BlockSpec((1, D), lambda i: (0, 0)),
        ],
        out_specs=pl.BlockSpec((BN, D), lambda i: (i, 0)),
        out_shape=jax.ShapeDtypeStruct((N, D), jnp.float32),
    )(x, w, b)


def _qkv_body(h_ref, w_ref, q_ref, k_ref, v_ref):
    qkv = jnp.dot(h_ref[...], w_ref[...], preferred_element_type=jnp.float32)
    q_ref[...] = qkv[:, :D] * SCALE
    k_ref[...] = qkv[:, D:2 * D]
    v_ref[...] = qkv[:, 2 * D:]


def _qkv(h, w):
    BN = 2000
    return pl.pallas_call(
        _qkv_body,
        grid=(N // BN,),
        in_specs=[
            pl.BlockSpec((BN, D), lambda i: (i, 0)),
            pl.BlockSpec((D, 3 * D), lambda i: (0, 0)),
        ],
        out_specs=[
            pl.BlockSpec((BN, D), lambda i: (i, 0)),
            pl.BlockSpec((BN, D), lambda i: (i, 0)),
            pl.BlockSpec((BN, D), lambda i: (i, 0)),
        ],
        out_shape=[
            jax.ShapeDtypeStruct((N, D), jnp.float32),
            jax.ShapeDtypeStruct((N, D), jnp.float32),
            jax.ShapeDtypeStruct((N, D), jnp.float32),
        ],
    )(h, w)


def _out_body(msg_ref, den_ref, h_ref, wo_ref, bo_ref, r_ref, o_ref):
    num = msg_ref[0] + msg_ref[1]
    den = den_ref[0, :, :H] + den_ref[1, :, :H]
    deninv = 1.0 / (den + 1e-9)
    den_big = jnp.dot(deninv, r_ref[...], preferred_element_type=jnp.float32)
    agg = num * den_big
    out = jax.nn.gelu(jnp.dot(agg, wo_ref[...],
                              preferred_element_type=jnp.float32)
                      + bo_ref[...])
    o_ref[...] = h_ref[...] + out


def _out_proj(sc_msg, sc_den, h, wo, bo, r):
    BN = 2000
    return pl.pallas_call(
        _out_body,
        grid=(N // BN,),
        in_specs=[
            pl.BlockSpec((NC, BN, D), lambda i: (0, i, 0)),
            pl.BlockSpec((NC, BN, DENW), lambda i: (0, i, 0)),
            pl.BlockSpec((BN, D), lambda i: (i, 0)),
            pl.BlockSpec((D, D), lambda i: (0, 0)),
            pl.BlockSpec((1, D), lambda i: (0, 0)),
            pl.BlockSpec((H, D), lambda i: (0, 0)),
        ],
        out_specs=pl.BlockSpec((BN, D), lambda i: (i, 0)),
        out_shape=jax.ShapeDtypeStruct((N, D), jnp.float32),
    )(sc_msg, sc_den, h, wo, bo, r)


@jax.jit
def kernel(x_vehicle, edge_index, edge_attr_v2v, W_in, b_in, W_ein, b_ein,
           Wq, Wk, Wv, We, Wo, bo):
    src = edge_index[0].astype(jnp.int32)
    dst = edge_index[1].astype(jnp.int32)

    w_ev = jnp.einsum("if,lfd->lid", W_ein, We)
    b_ev = jnp.einsum("f,lfd->ld", b_ein, We)
    r = jnp.repeat(jnp.eye(H, dtype=jnp.float32), DH, axis=1)

    ev = _ev_proj(edge_attr_v2v, w_ev, b_ev.reshape(L, 1, D))
    h = _embed(x_vehicle, W_in, b_in.reshape(1, D))
    for l in range(L):
        wqkv = jnp.concatenate([Wq[l], Wk[l], Wv[l]], axis=1)
        q, k, v = _qkv(h, wqkv)
        sc_msg, sc_den = _edge_attn(q, k, v, ev[l], src, dst)
        h = _out_proj(sc_msg, sc_den, h, Wo[l], bo[l].reshape(1, D), r)
    return h


# speedup vs baseline: 1.1613x; 1.1613x over previous
import functools

import jax
import jax.numpy as jnp
from jax import lax
from jax.experimental import pallas as pl
from jax.experimental.pallas import tpu as pltpu
from jax.experimental.pallas import tpu_sc as plsc

N = 10000
E = 320000
D_IN = 8
DE_IN = 10
D = 128
H = 4
DH = 32
L = 2
SCALE = 1.0 / (32.0 ** 0.5)

NC = 2
NS = 16
NW = NC * NS
EW = E // NW
CH = 40
NCHUNK = EW // CH
DENW = 8
NACC = 10112
TROWS = NACC // NS


def _edge_attn_body(q_hbm, k_hbm, v_hbm, ev_hbm, src_hbm, dst_hbm,
                    omsg_hbm, oden_hbm,
                    src_vA, dst_vA, qrowsA, krowsA, evrowsA, vrowsA, denbA,
                    src_vB, dst_vB, qrowsB, krowsB, evrowsB, vrowsB, denbB,
                    accm, accd, semA, semB, semI):
    c = lax.axis_index("c")
    s = lax.axis_index("s")
    wid = s * NC + c
    base = wid * EW

    zero16 = jnp.zeros((16,), jnp.float32)

    def zbody(r, carry):
        for kk in range(D // 16):
            qrowsA[r, pl.ds(kk * 16, 16)] = zero16
        return carry

    lax.fori_loop(0, CH, zbody, 0)
    iota16 = lax.iota(jnp.int32, 16)
    for kk in range(CH * DENW // 16):
        p = iota16 + kk * 16
        plsc.store_scatter(denbA, [p // DENW, p % DENW], zero16)
        plsc.store_scatter(denbB, [p // DENW, p % DENW], zero16)

    row0 = s * TROWS
    for z in range(15):
        pltpu.sync_copy(qrowsA, accm.at[pl.ds(row0 + z * CH, CH)])
        pltpu.sync_copy(denbA, accd.at[pl.ds(row0 + z * CH, CH)])
    rem = TROWS - 15 * CH
    pltpu.sync_copy(qrowsA.at[pl.ds(0, rem)], accm.at[pl.ds(row0 + 15 * CH, rem)])
    pltpu.sync_copy(denbA.at[pl.ds(0, rem)], accd.at[pl.ds(row0 + 15 * CH, rem)])
    plsc.subcore_barrier()

    lane15 = lax.iota(jnp.int32, 16) == 15
    full15 = jnp.full((16,), 15, jnp.int32)
    hconsts = [jnp.full((16,), h, jnp.int32) for h in range(H)]

    def load_idx(cn, src_v, dst_v):
        eb = base + cn * CH
        i1 = pltpu.async_copy(src_hbm.at[pl.ds(eb, CH)], src_v, semI)
        i2 = pltpu.async_copy(dst_hbm.at[pl.ds(eb, CH)], dst_v, semI)
        i1.wait()
        i2.wait()

    def issue(cn, src_v, dst_v, qrows, krows, vrows, evrows, sem):
        eb = base + cn * CH
        pltpu.async_copy(q_hbm.at[dst_v], qrows, sem)
        pltpu.async_copy(k_hbm.at[src_v], krows, sem)
        pltpu.async_copy(v_hbm.at[src_v], vrows, sem)
        pltpu.async_copy(ev_hbm.at[pl.ds(eb, CH)], evrows, sem)

    def drain(src_v, dst_v, qrows, krows, vrows, evrows, eb, sem):
        pltpu.make_async_copy(q_hbm.at[dst_v], qrows, sem).wait()
        pltpu.make_async_copy(k_hbm.at[src_v], krows, sem).wait()
        pltpu.make_async_copy(v_hbm.at[src_v], vrows, sem).wait()
        pltpu.make_async_copy(ev_hbm.at[pl.ds(eb, CH)], evrows, sem).wait()

    def compute(qrows, krows, vrows, evrows, denb):
        def edge1(jj, carry1):
            ph = []
            evs = []
            for cb in range(D // 16):
                qv = qrows[jj, pl.ds(cb * 16, 16)]
                kv = krows[jj, pl.ds(cb * 16, 16)]
                evv = evrows[jj, pl.ds(cb * 16, 16)]
                evs.append(evv)
                p = qv * (kv + evv)
                if cb % 2 == 0:
                    ph.append(p)
                else:
                    ph[cb // 2] = ph[cb // 2] + p
            jfull = jnp.full((16,), jj, jnp.int32)
            sh = []
            for h in range(H):
                cum = plsc.cumsum(ph[h])
                s_h = jnp.exp(cum[full15])
                sh.append(s_h)
                plsc.store_scatter(denb, [jfull, hconsts[h]], s_h, mask=lane15)
            for cb in range(D // 16):
                vv = vrows[jj, pl.ds(cb * 16, 16)]
                qrows[jj, pl.ds(cb * 16, 16)] = sh[cb // 2] * (vv + evs[cb])
            return carry1

        lax.fori_loop(0, CH, edge1, 0)

    load_idx(0, src_vA, dst_vA)
    issue(0, src_vA, dst_vA, qrowsA, krowsA, vrowsA, evrowsA, semA)
    load_idx(1, src_vB, dst_vB)
    issue(1, src_vB, dst_vB, qrowsB, krowsB, vrowsB, evrowsB, semB)

    def pair_body(i, carry):
        ebA = base + (2 * i) * CH
        drain(src_vA, dst_vA, qrowsA, krowsA, vrowsA, evrowsA, ebA, semA)
        compute(qrowsA, krowsA, vrowsA, evrowsA, denbA)
        pltpu.sync_copy(qrowsA, accm.at[dst_vA], add=True)
        pltpu.sync_copy(denbA, accd.at[dst_vA], add=True)
        cnA = jnp.minimum(2 * i + 2, NCHUNK - 1)
        load_idx(cnA, src_vA, dst_vA)
        issue(cnA, src_vA, dst_vA, qrowsA, krowsA, vrowsA, evrowsA, semA)
        ebB = base + (2 * i + 1) * CH
        drain(src_vB, dst_vB, qrowsB, krowsB, vrowsB, evrowsB, ebB, semB)
        compute(qrowsB, krowsB, vrowsB, evrowsB, denbB)
        pltpu.sync_copy(qrowsB, accm.at[dst_vB], add=True)
        pltpu.sync_copy(denbB, accd.at[dst_vB], add=True)
        cnB = jnp.minimum(2 * i + 3, NCHUNK - 1)
        load_idx(cnB, src_vB, dst_vB)
        issue(cnB, src_vB, dst_vB, qrowsB, krowsB, vrowsB, evrowsB, semB)
        return carry

    lax.fori_loop(0, NCHUNK // 2, pair_body, 0)
    ebL = base + (NCHUNK - 1) * CH
    drain(src_vA, dst_vA, qrowsA, krowsA, vrowsA, evrowsA, ebL, semA)
    drain(src_vB, dst_vB, qrowsB, krowsB, vrowsB, evrowsB, ebL, semB)
    plsc.subcore_barrier()
    pltpu.sync_copy(accm.at[pl.ds(row0, TROWS)],
                    omsg_hbm.at[c, pl.ds(row0, TROWS)])
    pltpu.sync_copy(accd.at[pl.ds(row0, TROWS)],
                    oden_hbm.at[c, pl.ds(row0, TROWS)])


def _edge_attn(q, k, v, ev, src, dst):
    mesh = plsc.VectorSubcoreMesh(core_axis_name="c", subcore_axis_name="s")
    f = pl.kernel(
        _edge_attn_body,
        mesh=mesh,
        out_type=[
            jax.ShapeDtypeStruct((NC, NACC, D), jnp.float32),
            jax.ShapeDtypeStruct((NC, NACC, DENW), jnp.float32),
        ],
        compiler_params=pltpu.CompilerParams(use_tc_tiling_on_sc=False,
                                             needs_layout_passes=False),
        scratch_types=[
            pltpu.VMEM((CH,), jnp.int32),
            pltpu.VMEM((CH,), jnp.int32),
            pltpu.VMEM((CH, D), jnp.float32),
            pltpu.VMEM((CH, D), jnp.float32),
            pltpu.VMEM((CH, D), jnp.float32),
            pltpu.VMEM((CH, D), jnp.float32),
            pltpu.VMEM((CH, DENW), jnp.float32),
            pltpu.VMEM((CH,), jnp.int32),
            pltpu.VMEM((CH,), jnp.int32),
            pltpu.VMEM((CH, D), jnp.float32),
            pltpu.VMEM((CH, D), jnp.float32),
            pltpu.VMEM((CH, D), jnp.float32),
            pltpu.VMEM((CH, D), jnp.float32),
            pltpu.VMEM((CH, DENW), jnp.float32),
            pltpu.VMEM_SHARED((NACC, D), jnp.float32),
            pltpu.VMEM_SHARED((NACC, DENW), jnp.float32),
            pltpu.SemaphoreType.DMA,
            pltpu.SemaphoreType.DMA,
            pltpu.SemaphoreType.DMA,
        ],
    )
    return f(q, k, v, ev, src, dst)


def _ev_body(e_ref, w_ref, b_ref, o_ref):
    o_ref[...] = (jnp.dot(e_ref[...], w_ref[0],
                          preferred_element_type=jnp.float32)
                  + b_ref[0])[None]


def _ev_proj(e_attr, w_ev, b_ev):
    BE = 2000
    return pl.pallas_call(
        _ev_body,
        grid=(L, E // BE),
        in_specs=[
            pl.BlockSpec((BE, DE_IN), lambda l, i: (i, 0)),
            pl.BlockSpec((1, DE_IN, D), lambda l, i: (l, 0, 0)),
            pl.BlockSpec((1, 1, D), lambda l, i: (l, 0, 0)),
        ],
        out_specs=pl.BlockSpec((1, BE, D), lambda l, i: (l, i, 0)),
        out_shape=jax.ShapeDtypeStruct((L, E, D), jnp.float32),
    )(e_attr, w_ev, b_ev)


def _embed_body(x_ref, w_ref, b_ref, o_ref):
    o_ref[...] = jnp.dot(x_ref[...], w_ref[...],
                         preferred_element_type=jnp.float32) + b_ref[...]


def _embed(x, w, b):
    BN = 2000
    return pl.pallas_call(
        _embed_body,
        grid=(N // BN,),
        in_specs=[
            pl.BlockSpec((BN, D_IN), lambda i: (i, 0)),
            pl.BlockSpec((D_IN, D), lambda i: (0, 0)),
            pl.BlockSpec((1, D), lambda i: (0, 0)),
        ],
        out_specs=pl.BlockSpec((BN, D), lambda i: (i, 0)),
        out_shape=jax.ShapeDtypeStruct((N, D), jnp.float32),
    )(x, w, b)


def _qkv_body(h_ref, w_ref, q_ref, k_ref, v_ref):
    qkv = jnp.dot(h_ref[...], w_ref[...], preferred_element_type=jnp.float32)
    q_ref[...] = qkv[:, :D] * SCALE
    k_ref[...] = qkv[:, D:2 * D]
    v_ref[...] = qkv[:, 2 * D:]


def _qkv(h, w):
    BN = 2000
    return pl.pallas_call(
        _qkv_body,
        grid=(N // BN,),
        in_specs=[
            pl.BlockSpec((BN, D), lambda i: (i, 0)),
            pl.BlockSpec((D, 3 * D), lambda i: (0, 0)),
        ],
        out_specs=[
            pl.BlockSpec((BN, D), lambda i: (i, 0)),
            pl.BlockSpec((BN, D), lambda i: (i, 0)),
            pl.BlockSpec((BN, D), lambda i: (i, 0)),
        ],
        out_shape=[
            jax.ShapeDtypeStruct((N, D), jnp.float32),
            jax.ShapeDtypeStruct((N, D), jnp.float32),
            jax.ShapeDtypeStruct((N, D), jnp.float32),
        ],
    )(h, w)


def _out_body(msg_ref, den_ref, h_ref, wo_ref, bo_ref, r_ref, o_ref):
    num = msg_ref[0] + msg_ref[1]
    den = den_ref[0, :, :H] + den_ref[1, :, :H]
    deninv = 1.0 / (den + 1e-9)
    den_big = jnp.dot(deninv, r_ref[...], preferred_element_type=jnp.float32)
    agg = num * den_big
    out = jax.nn.gelu(jnp.dot(agg, wo_ref[...],
                              preferred_element_type=jnp.float32)
                      + bo_ref[...])
    o_ref[...] = h_ref[...] + out


def _out_proj(sc_msg, sc_den, h, wo, bo, r):
    BN = 2000
    return pl.pallas_call(
        _out_body,
        grid=(N // BN,),
        in_specs=[
            pl.BlockSpec((NC, BN, D), lambda i: (0, i, 0)),
            pl.BlockSpec((NC, BN, DENW), lambda i: (0, i, 0)),
            pl.BlockSpec((BN, D), lambda i: (i, 0)),
            pl.BlockSpec((D, D), lambda i: (0, 0)),
            pl.BlockSpec((1, D), lambda i: (0, 0)),
            pl.BlockSpec((H, D), lambda i: (0, 0)),
        ],
        out_specs=pl.BlockSpec((BN, D), lambda i: (i, 0)),
        out_shape=jax.ShapeDtypeStruct((N, D), jnp.float32),
    )(sc_msg, sc_den, h, wo, bo, r)


@jax.jit
def kernel(x_vehicle, edge_index, edge_attr_v2v, W_in, b_in, W_ein, b_ein,
           Wq, Wk, Wv, We, Wo, bo):
    src = edge_index[0].astype(jnp.int32)
    dst = edge_index[1].astype(jnp.int32)

    w_ev = jnp.einsum("if,lfd->lid", W_ein, We)
    b_ev = jnp.einsum("f,lfd->ld", b_ein, We)
    r = jnp.repeat(jnp.eye(H, dtype=jnp.float32), DH, axis=1)

    ev = _ev_proj(edge_attr_v2v, w_ev, b_ev.reshape(L, 1, D))
    h = _embed(x_vehicle, W_in, b_in.reshape(1, D))
    for l in range(L):
        wqkv = jnp.concatenate([Wq[l], Wk[l], Wv[l]], axis=1)
        q, k, v = _qkv(h, wqkv)
        sc_msg, sc_den = _edge_attn(q, k, v, ev[l], src, dst)
        h = _out_proj(sc_msg, sc_den, h, Wo[l], bo[l].reshape(1, D), r)
    return h
